# baseline (speedup 1.0000x reference)
import jax
import jax.numpy as jnp
from jax import lax
from jax.experimental import pallas as pl
from jax.experimental.pallas import tpu as pltpu
from jax.experimental.pallas import tpu_sc as plsc

B = 1024
N_MF = 4096
N_GC = 8192
NSYN = 4
THETA = 0.75

L = 16
ROWS_PER_BLK = 16
GC_CHUNK = 1024
N_CHUNKS = N_GC // GC_CHUNK
ROWS_PER_CHUNK = GC_CHUNK * NSYN // 128


def _sc_body(x_hbm, idxr_hbm, wr_hbm, out_hbm,
             xtile, ibuf, wbuf, obuf, semi0, semi1, semo0, semo1):
    nc = 2
    wid = lax.axis_index("s") * nc + lax.axis_index("c")
    rows_per_worker = B // 32

    semi = [semi0, semi1]
    semo = [semo0, semo1]

    sidx = [
        [jnp.arange(0, 64, NSYN, dtype=jnp.int32) + (h * 64 + s) for s in range(NSYN)]
        for h in range(2)
    ]
    rrv = [jnp.full((L,), rr, dtype=jnp.int32) for rr in range(8)]

    def start_iw(c):
        slot = c % 2
        ci = pltpu.async_copy(
            idxr_hbm.at[pl.ds(c * ROWS_PER_CHUNK, ROWS_PER_CHUNK)],
            ibuf.at[slot],
            semi[slot],
        )
        cw = pltpu.async_copy(
            wr_hbm.at[pl.ds(c * ROWS_PER_CHUNK, ROWS_PER_CHUNK)],
            wbuf.at[slot],
            semi[slot],
        )
        return ci, cw

    for rb in range(rows_per_worker // ROWS_PER_BLK):
        row0 = wid * rows_per_worker + rb * ROWS_PER_BLK
        tr0 = row0 // 8
        iw_pending = start_iw(0)
        for t in range(ROWS_PER_BLK // 8):
            pltpu.sync_copy(x_hbm.at[tr0 + t], xtile.at[t])
        out_pending = [None, None]
        for c in range(N_CHUNKS):
            slot = c % 2
            iw_next = start_iw(c + 1) if c + 1 < N_CHUNKS else None
            iw_pending[0].wait()
            iw_pending[1].wait()
            iw_pending = iw_next
            if out_pending[slot] is not None:
                out_pending[slot].wait()

            @plsc.parallel_loop(0, ROWS_PER_CHUNK, 1)
            def row_body(j):
                tile_c = j // 4
                ib = ibuf.at[slot, j]
                wb = wbuf.at[slot, j]
                for h in range(2):
                    cin = pl.multiple_of((j % 4) * 32 + h * L, L)
                    iv = [plsc.load_gather(ib, [sidx[h][s]]) for s in range(NSYN)]
                    wv = [plsc.load_gather(wb, [sidx[h][s]]) for s in range(NSYN)]
                    ihi = [lax.shift_right_logical(iv[s], 7) for s in range(NSYN)]
                    ilo = [lax.bitwise_and(iv[s], 127) for s in range(NSYN)]

                    RGRP = 4
                    for r0 in range(0, ROWS_PER_BLK, RGRP):
                        gath = []
                        for r in range(r0, r0 + RGRP):
                            xr = xtile.at[r // 8]
                            gath.append(
                                [
                                    plsc.load_gather(xr, [ihi[s], rrv[r % 8], ilo[s]])
                                    for s in range(NSYN)
                                ]
                            )
                        for k, r in enumerate(range(r0, r0 + RGRP)):
                            ga = gath[k]
                            acc = (ga[0] * wv[0] + ga[1] * wv[1]) + (
                                ga[2] * wv[2] + ga[3] * wv[3]
                            )
                            obuf[slot, r // 8, tile_c, r % 8, pl.ds(cin, L)] = (
                                jnp.maximum(acc - THETA, 0.0)
                            )

            out_pending[slot] = pltpu.async_copy(
                obuf.at[slot],
                out_hbm.at[
                    pl.ds(tr0, ROWS_PER_BLK // 8),
                    pl.ds(c * (GC_CHUNK // 128), GC_CHUNK // 128),
                ],
                semo[slot],
            )
        for slot in range(2):
            if out_pending[slot] is not None:
                out_pending[slot].wait()


@jax.jit
def _mossy_granule_sc(x4, idx_r, w_r):
    mesh = plsc.VectorSubcoreMesh(core_axis_name="c", subcore_axis_name="s")
    kern = pl.kernel(
        _sc_body,
        out_type=jax.ShapeDtypeStruct((B // 8, N_GC // 128, 8, 128), jnp.float32),
        mesh=mesh,
        compiler_params=pltpu.CompilerParams(
            use_tc_tiling_on_sc=False, needs_layout_passes=False
        ),
        scratch_types=[
            pltpu.VMEM((2, N_MF // 128, 8, 128), jnp.float32),
            pltpu.VMEM((2, ROWS_PER_CHUNK, 128), jnp.int32),
            pltpu.VMEM((2, ROWS_PER_CHUNK, 128), jnp.float32),
            pltpu.VMEM(
                (2, ROWS_PER_BLK // 8, GC_CHUNK // 128, 8, 128), jnp.float32
            ),
            pltpu.SemaphoreType.DMA,
            pltpu.SemaphoreType.DMA,
            pltpu.SemaphoreType.DMA,
            pltpu.SemaphoreType.DMA,
        ],
    )
    y4 = kern(x4, idx_r, w_r)
    return y4.transpose(0, 2, 1, 3).reshape(B, N_GC)


def kernel(x, idx, W_conn):
    x4 = x.reshape(B // 8, 8, N_MF // 128, 128).transpose(0, 2, 1, 3)
    idx_r = idx.astype(jnp.int32).reshape(N_GC * NSYN // 128, 128)
    w_r = W_conn.astype(jnp.float32).reshape(N_GC * NSYN // 128, 128)
    return _mossy_granule_sc(x4, idx_r, w_r)


# speedup vs baseline: 1.3684x; 1.3684x over previous
import jax
import jax.numpy as jnp
from jax import lax
from jax.experimental import pallas as pl
from jax.experimental.pallas import tpu as pltpu
from jax.experimental.pallas import tpu_sc as plsc

B = 1024
N_MF = 4096
N_GC = 8192
NSYN = 4
THETA = 0.75

L = 16
ROWS_PER_BLK = 16
GC_CHUNK = 1024
N_CHUNKS = N_GC // GC_CHUNK
ROWS_PER_CHUNK = GC_CHUNK * NSYN // 128


def _sc_body(x_hbm, idxr_hbm, wr_hbm, out_hbm,
             xtile, ibuf, wbuf, obuf, semi0, semi1, semo0, semo1):
    nc = 2
    wid = lax.axis_index("s") * nc + lax.axis_index("c")
    rows_per_worker = B // 32

    semi = [semi0, semi1]
    semo = [semo0, semo1]

    sidx = [jnp.arange(0, 64, NSYN, dtype=jnp.int32) + s for s in range(NSYN)]
    rrv = [jnp.full((L,), rr, dtype=jnp.int32) for rr in range(8)]

    def start_iw(c):
        slot = c % 2
        ci = pltpu.async_copy(
            idxr_hbm.at[pl.ds(c * ROWS_PER_CHUNK, ROWS_PER_CHUNK)],
            ibuf.at[slot],
            semi[slot],
        )
        cw = pltpu.async_copy(
            wr_hbm.at[pl.ds(c * ROWS_PER_CHUNK, ROWS_PER_CHUNK)],
            wbuf.at[slot],
            semi[slot],
        )
        return ci, cw

    for rb in range(rows_per_worker // ROWS_PER_BLK):
        row0 = wid * rows_per_worker + rb * ROWS_PER_BLK
        tr0 = row0 // 8
        iw_pending = start_iw(0)
        for t in range(ROWS_PER_BLK // 8):
            pltpu.sync_copy(x_hbm.at[tr0 + t], xtile.at[t])
        out_pending = [None, None]
        for c in range(N_CHUNKS):
            slot = c % 2
            iw_next = start_iw(c + 1) if c + 1 < N_CHUNKS else None
            iw_pending[0].wait()
            iw_pending[1].wait()
            iw_pending = iw_next
            if out_pending[slot] is not None:
                out_pending[slot].wait()

            @plsc.parallel_loop(0, GC_CHUNK // L, 1)
            def group_body(g):
                tile_c = g // (128 // L)
                cin = pl.multiple_of((g % (128 // L)) * L, L)
                half = pl.multiple_of((g % 2) * 64, 64)
                ib = ibuf.at[slot, g // 2, pl.ds(half, 64)]
                wb = wbuf.at[slot, g // 2, pl.ds(half, 64)]
                iv = [plsc.load_gather(ib, [sidx[s]]) for s in range(NSYN)]
                wv = [plsc.load_gather(wb, [sidx[s]]) for s in range(NSYN)]
                ihi = [lax.shift_right_logical(iv[s], 7) for s in range(NSYN)]
                ilo = [lax.bitwise_and(iv[s], 127) for s in range(NSYN)]

                RGRP = 4
                for r0 in range(0, ROWS_PER_BLK, RGRP):
                    gath = []
                    for r in range(r0, r0 + RGRP):
                        xr = xtile.at[r // 8]
                        gath.append(
                            [
                                plsc.load_gather(xr, [ihi[s], rrv[r % 8], ilo[s]])
                                for s in range(NSYN)
                            ]
                        )
                    for k, r in enumerate(range(r0, r0 + RGRP)):
                        ga = gath[k]
                        acc = (ga[0] * wv[0] + ga[1] * wv[1]) + (
                            ga[2] * wv[2] + ga[3] * wv[3]
                        )
                        obuf[slot, r // 8, tile_c, r % 8, pl.ds(cin, L)] = (
                            jnp.maximum(acc - THETA, 0.0)
                        )

            out_pending[slot] = pltpu.async_copy(
                obuf.at[slot],
                out_hbm.at[
                    pl.ds(tr0, ROWS_PER_BLK // 8),
                    pl.ds(c * (GC_CHUNK // 128), GC_CHUNK // 128),
                ],
                semo[slot],
            )
        for slot in range(2):
            if out_pending[slot] is not None:
                out_pending[slot].wait()


@jax.jit
def _mossy_granule_sc(x4, idx_r, w_r):
    mesh = plsc.VectorSubcoreMesh(core_axis_name="c", subcore_axis_name="s")
    kern = pl.kernel(
        _sc_body,
        out_type=jax.ShapeDtypeStruct((B // 8, N_GC // 128, 8, 128), jnp.float32),
        mesh=mesh,
        compiler_params=pltpu.CompilerParams(
            use_tc_tiling_on_sc=False, needs_layout_passes=False
        ),
        scratch_types=[
            pltpu.VMEM((2, N_MF // 128, 8, 128), jnp.float32),
            pltpu.VMEM((2, ROWS_PER_CHUNK, 128), jnp.int32),
            pltpu.VMEM((2, ROWS_PER_CHUNK, 128), jnp.float32),
            pltpu.VMEM(
                (2, ROWS_PER_BLK // 8, GC_CHUNK // 128, 8, 128), jnp.float32
            ),
            pltpu.SemaphoreType.DMA,
            pltpu.SemaphoreType.DMA,
            pltpu.SemaphoreType.DMA,
            pltpu.SemaphoreType.DMA,
        ],
    )
    y4 = kern(x4, idx_r, w_r)
    return y4.transpose(0, 2, 1, 3).reshape(B, N_GC)


def kernel(x, idx, W_conn):
    x4 = x.reshape(B // 8, 8, N_MF // 128, 128).transpose(0, 2, 1, 3)
    idx_r = idx.astype(jnp.int32).reshape(N_GC * NSYN // 128, 128)
    w_r = W_conn.astype(jnp.float32).reshape(N_GC * NSYN // 128, 128)
    return _mossy_granule_sc(x4, idx_r, w_r)
